# Initial kernel scaffold; baseline (speedup 1.0000x reference)
import jax
import jax.numpy as jnp
from jax.experimental import pallas as pl


def kernel(x, w1, w2, w3, w4, w5, rw1, rw2, rw3, lin_w, lin_b):
    raise NotImplementedError("write your pallas kernel here")


# speedup vs baseline: 11.9805x; 11.9805x over previous
import functools

import jax
import jax.numpy as jnp
from jax import lax
from jax.experimental import pallas as pl
from jax.experimental.pallas import tpu as pltpu
from jax.experimental.pallas import tpu_sc as plsc

KNN = 8
EPS = 1e-5
B = 8
N = 2048
BN = B * N
ROWS = 256
NB = N // ROWS
H = lax.Precision.HIGHEST
F32 = jnp.float32
BF16 = jnp.bfloat16


def _lrelu(v):
    return jnp.where(v >= 0, v, 0.2 * v)


def _knn_front(xT, wrT):
    D = xT.shape[1]
    has_res = wrT is not None

    def body(xrow_ref, xfull_ref, *rest):
        if has_res:
            wr_ref, idx_ref, r_ref = rest
        else:
            (idx_ref,) = rest
        xr = xrow_ref[...]
        xf = xfull_ref[...]
        dot = lax.dot_general(xr.astype(BF16), xf.astype(BF16),
                              (((1,), (1,)), ((), ())),
                              preferred_element_type=F32)
        xxr = jnp.sum(xr * xr, axis=1, keepdims=True)
        ones = jnp.ones((1, D), F32)
        xxf = lax.dot_general(ones, xf * xf, (((1,), (1,)), ((), ())),
                              precision=H, preferred_element_type=F32)
        pd = 2.0 * dot - xxr - xxf
        iota = lax.broadcasted_iota(jnp.int32, (ROWS, N), 1)
        boff = pl.program_id(0) * N
        for r in range(KNN):
            m = jnp.max(pd, axis=1, keepdims=True)
            cand = jnp.where(pd == m, iota, N)
            j = jnp.min(cand, axis=1, keepdims=True)
            idx_ref[:, r:r + 1] = j + boff
            pd = jnp.where(cand == j, -jnp.inf, pd)
        if has_res:
            r_ref[...] = lax.dot_general(
                xr.astype(BF16), wr_ref[...], (((1,), (0,)), ((), ())),
                preferred_element_type=F32)

    in_specs = [
        pl.BlockSpec((ROWS, D), lambda b, i: (b * NB + i, 0)),
        pl.BlockSpec((N, D), lambda b, i: (b, 0)),
    ]
    operands = [xT, xT]
    out_shape = [jax.ShapeDtypeStruct((BN, KNN), jnp.int32)]
    out_specs = [pl.BlockSpec((ROWS, KNN), lambda b, i: (b * NB + i, 0))]
    if has_res:
        O = wrT.shape[1]
        in_specs.append(pl.BlockSpec((D, O), lambda b, i: (0, 0)))
        operands.append(wrT.astype(BF16))
        out_shape.append(jax.ShapeDtypeStruct((BN, O), F32))
        out_specs.append(pl.BlockSpec((ROWS, O), lambda b, i: (b * NB + i, 0)))
    res = pl.pallas_call(
        body, grid=(B, NB), in_specs=in_specs, out_specs=out_specs,
        out_shape=out_shape)(*operands)
    return res if has_res else (res[0], None)


def _sc_gather(xT, idxf):
    D = xT.shape[1]
    info = plsc.get_sparse_core_info()
    NC, NS = info.num_cores, info.num_subcores
    NW = NC * NS
    P = BN // NW
    C = 16
    NCH = P // C
    mesh = plsc.VectorSubcoreMesh(core_axis_name="c", subcore_axis_name="s")

    @functools.partial(
        pl.kernel, mesh=mesh,
        out_type=jax.ShapeDtypeStruct((BN * KNN, D), F32),
        scratch_types=[
            pltpu.VMEM((C * KNN,), jnp.int32),
            pltpu.VMEM((C * KNN, D), F32),
            pltpu.SemaphoreType.DMA,
        ])
    def sc_fn(x_hbm, idx_hbm, nb_hbm, idx_v, rows_v, sem):
        wid = lax.axis_index("s") * NC + lax.axis_index("c")
        base0 = wid * P * KNN

        def chunk(ci, _):
            base = base0 + ci * (C * KNN)
            pltpu.sync_copy(idx_hbm.at[pl.ds(base, C * KNN)], idx_v)
            pltpu.async_copy(x_hbm.at[idx_v], rows_v, sem).wait()
            pltpu.sync_copy(rows_v, nb_hbm.at[pl.ds(base, C * KNN)])
            return 0

        lax.fori_loop(0, NCH, chunk, 0)

    return sc_fn(xT, idxf)


def _edge_conv(nb, xT, w2dT):
    D = xT.shape[1]
    O = w2dT.shape[1]

    def body(nb_ref, x_ref, w_ref, m_ref, s_ref):
        center = x_ref[...]
        crep = jnp.broadcast_to(center[:, None, :], (ROWS, KNN, D))
        crep = crep.reshape(ROWS * KNN, D)
        diff = nb_ref[...] - crep
        feat = jnp.concatenate(
            [diff.astype(BF16), crep.astype(BF16)], axis=1)
        e = lax.dot_general(feat, w_ref[...], (((1,), (0,)), ((), ())),
                            preferred_element_type=F32)
        m_ref[...] = jnp.max(e.reshape(ROWS, KNN, O), axis=1)
        bs = jnp.sum(e, axis=0, keepdims=True)
        bs2 = jnp.sum(e * e, axis=0, keepdims=True)
        upd = jnp.concatenate([bs, bs2, jnp.zeros((6, O), F32)], axis=0)
        first = (pl.program_id(0) == 0) & (pl.program_id(1) == 0)

        @pl.when(first)
        def _():
            s_ref[...] = upd

        @pl.when(jnp.logical_not(first))
        def _():
            s_ref[...] = s_ref[...] + upd

    return pl.pallas_call(
        body, grid=(B, NB),
        in_specs=[
            pl.BlockSpec((ROWS * KNN, D), lambda b, i: (b * NB + i, 0)),
            pl.BlockSpec((ROWS, D), lambda b, i: (b * NB + i, 0)),
            pl.BlockSpec((2 * D, O), lambda b, i: (0, 0)),
        ],
        out_specs=[pl.BlockSpec((ROWS, O), lambda b, i: (b * NB + i, 0)),
                   pl.BlockSpec((8, O), lambda b, i: (0, 0))],
        out_shape=[jax.ShapeDtypeStruct((BN, O), F32),
                   jax.ShapeDtypeStruct((8, O), F32)],
    )(nb, xT, w2dT)


def _edge_update(M, sacc, resT):
    O = M.shape[1]
    has_res = resT is not None
    cnt = float(BN * KNN)

    def body(m_ref, s_ref, *rest):
        if has_res:
            r_ref, o_ref = rest
        else:
            (o_ref,) = rest
        s = s_ref[...]
        mean = s[0:1] / cnt
        inv = lax.rsqrt(s[1:2] / cnt - mean * mean + EPS)
        out = _lrelu((m_ref[...] - mean) * inv)
        if has_res:
            out = out + r_ref[...]
        o_ref[...] = out

    specs = [
        pl.BlockSpec((ROWS, O), lambda b, i: (b * NB + i, 0)),
        pl.BlockSpec((8, O), lambda b, i: (0, 0)),
    ]
    operands = [M, sacc]
    if has_res:
        specs.append(pl.BlockSpec((ROWS, O), lambda b, i: (b * NB + i, 0)))
        operands.append(resT)
    return pl.pallas_call(
        body, grid=(B, NB), in_specs=specs,
        out_specs=pl.BlockSpec((ROWS, O), lambda b, i: (b * NB + i, 0)),
        out_shape=jax.ShapeDtypeStruct((BN, O), F32))(*operands)


def _conv5(x1, x2, x3, x4, w51T, w52T, w53T, w54T):
    CO = w51T.shape[1]

    def body(x1_ref, x2_ref, x3_ref, x4_ref, wa_ref, wb_ref, wc_ref, wd_ref,
             o_ref, s_ref):
        def bdot(xr, wr):
            return lax.dot_general(xr[...].astype(BF16), wr[...],
                                   (((1,), (0,)), ((), ())),
                                   preferred_element_type=F32)
        o = bdot(x1_ref, wa_ref) + bdot(x2_ref, wb_ref)
        o = o + bdot(x3_ref, wc_ref) + bdot(x4_ref, wd_ref)
        o_ref[...] = o
        bs = jnp.sum(o, axis=0, keepdims=True)
        bs2 = jnp.sum(o * o, axis=0, keepdims=True)
        upd = jnp.concatenate([bs, bs2, jnp.zeros((6, CO), F32)], axis=0)
        first = (pl.program_id(0) == 0) & (pl.program_id(1) == 0)

        @pl.when(first)
        def _():
            s_ref[...] = upd

        @pl.when(jnp.logical_not(first))
        def _():
            s_ref[...] = s_ref[...] + upd

    def xspec(d):
        return pl.BlockSpec((ROWS, d), lambda b, i: (b * NB + i, 0))

    def wspec(d):
        return pl.BlockSpec((d, CO), lambda b, i: (0, 0))

    return pl.pallas_call(
        body, grid=(B, NB),
        in_specs=[xspec(128), xspec(128), xspec(256), xspec(512),
                  wspec(128), wspec(128), wspec(256), wspec(512)],
        out_specs=[pl.BlockSpec((ROWS, CO), lambda b, i: (b * NB + i, 0)),
                   pl.BlockSpec((8, CO), lambda b, i: (0, 0))],
        out_shape=[jax.ShapeDtypeStruct((BN, CO), F32),
                   jax.ShapeDtypeStruct((8, CO), F32)],
    )(x1, x2, x3, x4, w51T.astype(BF16), w52T.astype(BF16),
      w53T.astype(BF16), w54T.astype(BF16))


def _norm_pool(out, sacc):
    CO = out.shape[1]
    cnt = float(BN)

    def body(o_ref, s_ref, p_ref):
        s = s_ref[...]
        mean = s[0:1] / cnt
        inv = lax.rsqrt(s[1:2] / cnt - mean * mean + EPS)
        xcn = _lrelu((o_ref[...] - mean) * inv)
        bm = jnp.max(xcn, axis=0, keepdims=True)
        bs = jnp.sum(xcn, axis=0, keepdims=True)
        z = jnp.zeros((6, CO), F32)
        i = pl.program_id(1)

        @pl.when(i == 0)
        def _():
            p_ref[...] = jnp.concatenate([bm, bs, z], axis=0)[None]

        @pl.when(i != 0)
        def _():
            prev = p_ref[0]
            p_ref[...] = jnp.concatenate(
                [jnp.maximum(prev[0:1], bm), prev[1:2] + bs, z], axis=0)[None]

    return pl.pallas_call(
        body, grid=(B, NB),
        in_specs=[pl.BlockSpec((ROWS, CO), lambda b, i: (b * NB + i, 0)),
                  pl.BlockSpec((8, CO), lambda b, i: (0, 0))],
        out_specs=pl.BlockSpec((1, 8, CO), lambda b, i: (b, 0, 0)),
        out_shape=jax.ShapeDtypeStruct((B, 8, CO), F32))(out, sacc)


def _head(pooled, lin_wT, lin_b2):
    CO = pooled.shape[2]
    FD = lin_wT.shape[0]
    CB = 512
    NCB = lin_wT.shape[1] // CB

    def body(p_ref, w_ref, b_ref, y_ref):
        p1 = p_ref[:, 0, :].astype(BF16)
        p2 = (p_ref[:, 1, :] / float(N)).astype(BF16)
        w = w_ref[...]
        y = lax.dot_general(p1, w[:CO], (((1,), (0,)), ((), ())),
                            preferred_element_type=F32)
        y += lax.dot_general(p2, w[CO:], (((1,), (0,)), ((), ())),
                             preferred_element_type=F32)
        y += b_ref[...]
        mean = jnp.mean(y, axis=0, keepdims=True)
        inv = lax.rsqrt(jnp.mean(y * y, axis=0, keepdims=True)
                        - mean * mean + EPS)
        y_ref[...] = jnp.maximum((y - mean) * inv, 0.0)

    return pl.pallas_call(
        body, grid=(NCB,),
        in_specs=[pl.BlockSpec((B, 8, CO), lambda j: (0, 0, 0)),
                  pl.BlockSpec((FD, CB), lambda j: (0, j)),
                  pl.BlockSpec((1, CB), lambda j: (0, j))],
        out_specs=pl.BlockSpec((B, CB), lambda j: (0, j)),
        out_shape=jax.ShapeDtypeStruct((B, lin_wT.shape[1]), F32),
    )(pooled, lin_wT.astype(BF16), lin_b2)


def _edge_layer(xT, w, d, wrT, pad_to=None):
    wa, wb = w[:, :d], w[:, d:]
    w2dT = jnp.concatenate([wa.T, wb.T], axis=0)
    if pad_to is not None:
        zpad = jnp.zeros((pad_to - d, w.shape[0]), w.dtype)
        w2dT = jnp.concatenate([wa.T, zpad, wb.T, zpad], axis=0)
    idx, resT = _knn_front(xT, wrT)
    nb = _sc_gather(xT, idx.reshape(BN * KNN))
    M, sacc = _edge_conv(nb, xT, w2dT.astype(BF16))
    return _edge_update(M, sacc, resT)


def kernel(x, w1, w2, w3, w4, w5, rw1, rw2, rw3, lin_w, lin_b):
    xT = jnp.transpose(x, (0, 2, 1)).reshape(BN, 3)
    xT = jnp.pad(xT, ((0, 0), (0, 125)))
    x1 = _edge_layer(xT, w1, 3, None, pad_to=128)
    x2 = _edge_layer(x1, w2, 128, rw1.T)
    x3 = _edge_layer(x2, w3, 128, rw2.T)
    x4 = _edge_layer(x3, w4, 256, rw3.T)
    out, sacc = _conv5(x1, x2, x3, x4,
                       w5[:, :128].T, w5[:, 128:256].T,
                       w5[:, 256:512].T, w5[:, 512:].T)
    pooled = _norm_pool(out, sacc)
    return _head(pooled, lin_w.T, lin_b[None, :])


# speedup vs baseline: 13.9491x; 1.1643x over previous
import functools

import jax
import jax.numpy as jnp
from jax import lax
from jax.experimental import pallas as pl
from jax.experimental.pallas import tpu as pltpu
from jax.experimental.pallas import tpu_sc as plsc

KNN = 8
EPS = 1e-5
B = 8
N = 2048
BN = B * N
ROWS = 256
NB = N // ROWS
H = lax.Precision.HIGHEST
F32 = jnp.float32
BF16 = jnp.bfloat16


def _lrelu(v):
    return jnp.where(v >= 0, v, 0.2 * v)


def _knn_front(xT, wrT):
    D = xT.shape[1]
    has_res = wrT is not None

    def body(xrow_ref, xfull_ref, *rest):
        if has_res:
            wr_ref, idx_ref, r_ref = rest
        else:
            (idx_ref,) = rest
        xr = xrow_ref[...]
        xf = xfull_ref[...]
        dot = lax.dot_general(xr.astype(BF16), xf.astype(BF16),
                              (((1,), (1,)), ((), ())),
                              preferred_element_type=F32)
        xxr = jnp.sum(xr * xr, axis=1, keepdims=True)
        ones = jnp.ones((1, D), F32)
        xxf = lax.dot_general(ones, xf * xf, (((1,), (1,)), ((), ())),
                              precision=H, preferred_element_type=F32)
        pd = 2.0 * dot - xxr - xxf
        iota = lax.broadcasted_iota(jnp.int32, (ROWS, N), 1).astype(F32)
        boff = pl.program_id(0) * N
        big = jnp.float32(N)
        for r in range(KNN):
            m = jnp.max(pd, axis=1, keepdims=True)
            cand = jnp.where(pd == m, iota, big)
            j = jnp.min(cand, axis=1, keepdims=True)
            idx_ref[:, r:r + 1] = j.astype(jnp.int32) + boff
            pd = jnp.where(cand == j, -jnp.inf, pd)
        if has_res:
            r_ref[...] = lax.dot_general(
                xr.astype(BF16), wr_ref[...], (((1,), (0,)), ((), ())),
                preferred_element_type=F32)

    in_specs = [
        pl.BlockSpec((ROWS, D), lambda b, i: (b * NB + i, 0)),
        pl.BlockSpec((N, D), lambda b, i: (b, 0)),
    ]
    operands = [xT, xT]
    out_shape = [jax.ShapeDtypeStruct((BN, KNN), jnp.int32)]
    out_specs = [pl.BlockSpec((ROWS, KNN), lambda b, i: (b * NB + i, 0))]
    if has_res:
        O = wrT.shape[1]
        in_specs.append(pl.BlockSpec((D, O), lambda b, i: (0, 0)))
        operands.append(wrT.astype(BF16))
        out_shape.append(jax.ShapeDtypeStruct((BN, O), F32))
        out_specs.append(pl.BlockSpec((ROWS, O), lambda b, i: (b * NB + i, 0)))
    res = pl.pallas_call(
        body, grid=(B, NB), in_specs=in_specs, out_specs=out_specs,
        out_shape=out_shape)(*operands)
    return res if has_res else (res[0], None)


def _sc_gather(xT, idxf):
    D = xT.shape[1]
    info = plsc.get_sparse_core_info()
    NC, NS = info.num_cores, info.num_subcores
    NW = NC * NS
    P = BN // NW
    C = 16
    NCH = P // C
    mesh = plsc.VectorSubcoreMesh(core_axis_name="c", subcore_axis_name="s")

    CR = C * KNN

    @functools.partial(
        pl.kernel, mesh=mesh,
        out_type=jax.ShapeDtypeStruct((BN * KNN, D), F32),
        scratch_types=[
            pltpu.VMEM((NCH, CR), jnp.int32),
            pltpu.VMEM((CR, D), F32),
            pltpu.VMEM((CR, D), F32),
            pltpu.SemaphoreType.DMA,
            pltpu.SemaphoreType.DMA,
            pltpu.SemaphoreType.DMA,
            pltpu.SemaphoreType.DMA,
        ])
    def sc_fn(x_hbm, idx_hbm, nb_hbm, idx_all, rows0, rows1,
              gsem0, gsem1, wsem0, wsem1):
        wid = lax.axis_index("s") * NC + lax.axis_index("c")
        base0 = wid * P * KNN
        rows = (rows0, rows1)
        gsem = (gsem0, gsem1)
        wsem = (wsem0, wsem1)

        pltpu.sync_copy(idx_hbm.at[pl.ds(wid * NCH, NCH)], idx_all)
        pltpu.async_copy(x_hbm.at[idx_all.at[0]], rows0, gsem0)

        def chunk(ci, _):
            base = base0 + ci * CR
            for p in range(2):
                @pl.when(lax.rem(ci, 2) == p)
                def _():
                    pltpu.make_async_copy(
                        x_hbm.at[idx_all.at[ci]], rows[p], gsem[p]).wait()

                    @pl.when(ci + 1 < NCH)
                    def _():
                        @pl.when(ci > 0)
                        def _():
                            pltpu.make_async_copy(
                                rows[1 - p], nb_hbm.at[pl.ds(base, CR)],
                                wsem[1 - p]).wait()
                        pltpu.async_copy(
                            x_hbm.at[idx_all.at[ci + 1]], rows[1 - p],
                            gsem[1 - p])
                    pltpu.async_copy(
                        rows[p], nb_hbm.at[pl.ds(base, CR)], wsem[p])
            return 0

        lax.fori_loop(0, NCH, chunk, 0)
        pltpu.make_async_copy(rows0, nb_hbm.at[pl.ds(base0, CR)],
                              wsem0).wait()
        pltpu.make_async_copy(rows1, nb_hbm.at[pl.ds(base0, CR)],
                              wsem1).wait()

    return sc_fn(xT, idxf.reshape(BN * KNN // CR, CR))


def _edge_conv(nb, xT, w2dT):
    D = xT.shape[1]
    O = w2dT.shape[1]
    W = w2dT.shape[0]
    d_lo = W // 2

    def body(nb_ref, x_ref, w_ref, m_ref, s_ref):
        center = x_ref[...]
        crep = jnp.broadcast_to(center[:, None, :], (ROWS, KNN, D))
        crep = crep.reshape(ROWS * KNN, D)
        diff = nb_ref[...] - crep
        feat = jnp.concatenate(
            [diff[:, :d_lo].astype(BF16),
             crep[:, :W - d_lo].astype(BF16)], axis=1)
        e = lax.dot_general(feat, w_ref[...], (((1,), (0,)), ((), ())),
                            preferred_element_type=F32)
        m_ref[...] = jnp.max(e.reshape(ROWS, KNN, O), axis=1)
        bs = jnp.sum(e, axis=0, keepdims=True)
        bs2 = jnp.sum(e * e, axis=0, keepdims=True)
        upd = jnp.concatenate([bs, bs2, jnp.zeros((6, O), F32)], axis=0)
        first = (pl.program_id(0) == 0) & (pl.program_id(1) == 0)

        @pl.when(first)
        def _():
            s_ref[...] = upd

        @pl.when(jnp.logical_not(first))
        def _():
            s_ref[...] = s_ref[...] + upd

    return pl.pallas_call(
        body, grid=(B, NB),
        in_specs=[
            pl.BlockSpec((ROWS * KNN, D), lambda b, i: (b * NB + i, 0)),
            pl.BlockSpec((ROWS, D), lambda b, i: (b * NB + i, 0)),
            pl.BlockSpec((W, O), lambda b, i: (0, 0)),
        ],
        out_specs=[pl.BlockSpec((ROWS, O), lambda b, i: (b * NB + i, 0)),
                   pl.BlockSpec((8, O), lambda b, i: (0, 0))],
        out_shape=[jax.ShapeDtypeStruct((BN, O), F32),
                   jax.ShapeDtypeStruct((8, O), F32)],
    )(nb, xT, w2dT)


def _edge_update(M, sacc, resT):
    O = M.shape[1]
    has_res = resT is not None
    cnt = float(BN * KNN)

    def body(m_ref, s_ref, *rest):
        if has_res:
            r_ref, o_ref = rest
        else:
            (o_ref,) = rest
        s = s_ref[...]
        mean = s[0:1] / cnt
        inv = lax.rsqrt(s[1:2] / cnt - mean * mean + EPS)
        out = _lrelu((m_ref[...] - mean) * inv)
        if has_res:
            out = out + r_ref[...]
        o_ref[...] = out

    specs = [
        pl.BlockSpec((ROWS, O), lambda b, i: (b * NB + i, 0)),
        pl.BlockSpec((8, O), lambda b, i: (0, 0)),
    ]
    operands = [M, sacc]
    if has_res:
        specs.append(pl.BlockSpec((ROWS, O), lambda b, i: (b * NB + i, 0)))
        operands.append(resT)
    return pl.pallas_call(
        body, grid=(B, NB), in_specs=specs,
        out_specs=pl.BlockSpec((ROWS, O), lambda b, i: (b * NB + i, 0)),
        out_shape=jax.ShapeDtypeStruct((BN, O), F32))(*operands)


def _conv5(x1, x2, x3, x4, w51T, w52T, w53T, w54T):
    CO = w51T.shape[1]

    def body(x1_ref, x2_ref, x3_ref, x4_ref, wa_ref, wb_ref, wc_ref, wd_ref,
             o_ref, s_ref):
        def bdot(xr, wr):
            return lax.dot_general(xr[...].astype(BF16), wr[...],
                                   (((1,), (0,)), ((), ())),
                                   preferred_element_type=F32)
        o = bdot(x1_ref, wa_ref) + bdot(x2_ref, wb_ref)
        o = o + bdot(x3_ref, wc_ref) + bdot(x4_ref, wd_ref)
        o_ref[...] = o
        bs = jnp.sum(o, axis=0, keepdims=True)
        bs2 = jnp.sum(o * o, axis=0, keepdims=True)
        upd = jnp.concatenate([bs, bs2, jnp.zeros((6, CO), F32)], axis=0)
        first = (pl.program_id(0) == 0) & (pl.program_id(1) == 0)

        @pl.when(first)
        def _():
            s_ref[...] = upd

        @pl.when(jnp.logical_not(first))
        def _():
            s_ref[...] = s_ref[...] + upd

    def xspec(d):
        return pl.BlockSpec((ROWS, d), lambda b, i: (b * NB + i, 0))

    def wspec(d):
        return pl.BlockSpec((d, CO), lambda b, i: (0, 0))

    return pl.pallas_call(
        body, grid=(B, NB),
        in_specs=[xspec(128), xspec(128), xspec(256), xspec(512),
                  wspec(128), wspec(128), wspec(256), wspec(512)],
        out_specs=[pl.BlockSpec((ROWS, CO), lambda b, i: (b * NB + i, 0)),
                   pl.BlockSpec((8, CO), lambda b, i: (0, 0))],
        out_shape=[jax.ShapeDtypeStruct((BN, CO), F32),
                   jax.ShapeDtypeStruct((8, CO), F32)],
    )(x1, x2, x3, x4, w51T.astype(BF16), w52T.astype(BF16),
      w53T.astype(BF16), w54T.astype(BF16))


def _norm_pool(out, sacc):
    CO = out.shape[1]
    cnt = float(BN)

    def body(o_ref, s_ref, p_ref):
        s = s_ref[...]
        mean = s[0:1] / cnt
        inv = lax.rsqrt(s[1:2] / cnt - mean * mean + EPS)
        xcn = _lrelu((o_ref[...] - mean) * inv)
        bm = jnp.max(xcn, axis=0, keepdims=True)
        bs = jnp.sum(xcn, axis=0, keepdims=True)
        z = jnp.zeros((6, CO), F32)
        i = pl.program_id(1)

        @pl.when(i == 0)
        def _():
            p_ref[...] = jnp.concatenate([bm, bs, z], axis=0)[None]

        @pl.when(i != 0)
        def _():
            prev = p_ref[0]
            p_ref[...] = jnp.concatenate(
                [jnp.maximum(prev[0:1], bm), prev[1:2] + bs, z], axis=0)[None]

    return pl.pallas_call(
        body, grid=(B, NB),
        in_specs=[pl.BlockSpec((ROWS, CO), lambda b, i: (b * NB + i, 0)),
                  pl.BlockSpec((8, CO), lambda b, i: (0, 0))],
        out_specs=pl.BlockSpec((1, 8, CO), lambda b, i: (b, 0, 0)),
        out_shape=jax.ShapeDtypeStruct((B, 8, CO), F32))(out, sacc)


def _head(pooled, lin_wT, lin_b2):
    CO = pooled.shape[2]
    FD = lin_wT.shape[0]
    CB = 512
    NCB = lin_wT.shape[1] // CB

    def body(p_ref, w_ref, b_ref, y_ref):
        p1 = p_ref[:, 0, :].astype(BF16)
        p2 = (p_ref[:, 1, :] / float(N)).astype(BF16)
        w = w_ref[...]
        y = lax.dot_general(p1, w[:CO], (((1,), (0,)), ((), ())),
                            preferred_element_type=F32)
        y += lax.dot_general(p2, w[CO:], (((1,), (0,)), ((), ())),
                             preferred_element_type=F32)
        y += b_ref[...]
        mean = jnp.mean(y, axis=0, keepdims=True)
        inv = lax.rsqrt(jnp.mean(y * y, axis=0, keepdims=True)
                        - mean * mean + EPS)
        y_ref[...] = jnp.maximum((y - mean) * inv, 0.0)

    return pl.pallas_call(
        body, grid=(NCB,),
        in_specs=[pl.BlockSpec((B, 8, CO), lambda j: (0, 0, 0)),
                  pl.BlockSpec((FD, CB), lambda j: (0, j)),
                  pl.BlockSpec((1, CB), lambda j: (0, j))],
        out_specs=pl.BlockSpec((B, CB), lambda j: (0, j)),
        out_shape=jax.ShapeDtypeStruct((B, lin_wT.shape[1]), F32),
    )(pooled, lin_wT.astype(BF16), lin_b2)


def _edge_layer(xT, w, d, wrT, pad_to=None):
    wa, wb = w[:, :d], w[:, d:]
    w2dT = jnp.concatenate([wa.T, wb.T], axis=0)
    if pad_to is not None:
        zpad = jnp.zeros((pad_to - d, w.shape[0]), w.dtype)
        w2dT = jnp.concatenate([wa.T, zpad, wb.T, zpad], axis=0)
    idx, resT = _knn_front(xT, wrT)
    nb = _sc_gather(xT, idx.reshape(BN * KNN))
    M, sacc = _edge_conv(nb, xT, w2dT.astype(BF16))
    return _edge_update(M, sacc, resT)


def kernel(x, w1, w2, w3, w4, w5, rw1, rw2, rw3, lin_w, lin_b):
    xT = jnp.transpose(x, (0, 2, 1)).reshape(BN, 3)
    xT = jnp.pad(xT, ((0, 0), (0, 125)))
    x1 = _edge_layer(xT, w1, 3, None, pad_to=128)
    x2 = _edge_layer(x1, w2, 128, rw1.T)
    x3 = _edge_layer(x2, w3, 128, rw2.T)
    x4 = _edge_layer(x3, w4, 256, rw3.T)
    out, sacc = _conv5(x1, x2, x3, x4,
                       w5[:, :128].T, w5[:, 128:256].T,
                       w5[:, 256:512].T, w5[:, 512:].T)
    pooled = _norm_pool(out, sacc)
    return _head(pooled, lin_w.T, lin_b[None, :])
